# Initial kernel scaffold; baseline (speedup 1.0000x reference)
import jax
import jax.numpy as jnp
from jax.experimental import pallas as pl


def kernel(vertex, adj, weights, bias):
    raise NotImplementedError("write your pallas kernel here")


# speedup vs baseline: 1.0408x; 1.0408x over previous
import jax
import jax.numpy as jnp
from jax.experimental import pallas as pl
from jax.experimental.pallas import tpu as pltpu

_BM = 400


def _gc_kernel(vertex_ref, weights_ref, bias_ref, adj_ref, out_ref, support_ref):
    @pl.when(pl.program_id(0) == 0)
    def _():
        s = jnp.dot(
            vertex_ref[...], weights_ref[...], preferred_element_type=jnp.float32
        )
        support_ref[...] = s.astype(jnp.bfloat16)

    acc = jnp.dot(
        adj_ref[...].astype(jnp.bfloat16),
        support_ref[...],
        preferred_element_type=jnp.float32,
    )
    out_ref[...] = acc + bias_ref[...]


def kernel(vertex, adj, weights, bias):
    n, din = vertex.shape
    dout = weights.shape[1]
    bias2 = bias.reshape(1, dout)
    return pl.pallas_call(
        _gc_kernel,
        grid=(n // _BM,),
        in_specs=[
            pl.BlockSpec((n, din), lambda i: (0, 0)),
            pl.BlockSpec((din, dout), lambda i: (0, 0)),
            pl.BlockSpec((1, dout), lambda i: (0, 0)),
            pl.BlockSpec((_BM, n), lambda i: (i, 0)),
        ],
        out_specs=pl.BlockSpec((_BM, dout), lambda i: (i, 0)),
        out_shape=jax.ShapeDtypeStruct((n, dout), jnp.float32),
        scratch_shapes=[pltpu.VMEM((n, dout), jnp.bfloat16)],
    )(vertex, weights, bias2, adj)
